# Initial kernel scaffold; baseline (speedup 1.0000x reference)
import jax
import jax.numpy as jnp
from jax.experimental import pallas as pl


def kernel(x, edge_index, coeffs):
    raise NotImplementedError("write your pallas kernel here")


# speedup vs baseline: 6.3128x; 6.3128x over previous
import functools

import jax
import jax.numpy as jnp
from jax import lax
from jax.experimental import pallas as pl
from jax.experimental.pallas import tpu as pltpu
from jax.experimental.pallas import tpu_sc as plsc

ALPHA = 1.0
BETA = 1.0

NC = 2
NS = 16
NW = NC * NS
LANES = 16
BATCH = 128


def _sc_mesh():
    return plsc.VectorSubcoreMesh(core_axis_name="c", subcore_axis_name="s")


def _make_deg_kernel(nrows, epw_pad):
    nzero = nrows // LANES
    nit = epw_pad // LANES

    @functools.partial(
        pl.kernel,
        out_type=jax.ShapeDtypeStruct((NW, nrows), jnp.float32),
        mesh=_sc_mesh(),
        scratch_types=[
            pltpu.VMEM((epw_pad,), jnp.int32),
            pltpu.VMEM((nrows,), jnp.float32),
        ],
        compiler_params=pltpu.CompilerParams(needs_layout_passes=False),
    )
    def deg_kernel(dst_hbm, out_hbm, dst_v, deg_v):
        c = lax.axis_index("c")
        s = lax.axis_index("s")
        wid = s * NC + c
        pltpu.sync_copy(dst_hbm.at[wid], dst_v)

        def zero_body(i, carry):
            deg_v[pl.ds(i * LANES, LANES)] = jnp.zeros((LANES,), jnp.float32)
            return carry

        lax.fori_loop(0, nzero, zero_body, 0)

        ones = jnp.ones((LANES,), jnp.float32)

        def body(i, carry):
            idx = dst_v[pl.ds(i * LANES, LANES)]
            plsc.addupdate_scatter(deg_v, [idx], ones)
            return carry

        lax.fori_loop(0, nit, body, 0)
        pltpu.sync_copy(deg_v, out_hbm.at[wid])

    return deg_kernel


def _make_spmv_kernel(n_nodes, nrows, d, nb):
    rows_per_sub = nrows // NS
    chunks = []
    off = 0
    while off < rows_per_sub:
        sz = min(BATCH, rows_per_sub - off)
        chunks.append((off, sz))
        off += sz

    @functools.partial(
        pl.kernel,
        out_type=jax.ShapeDtypeStruct((NC, nrows, d), jnp.float32),
        mesh=_sc_mesh(),
        scratch_types=[
            pltpu.VMEM((nb, BATCH), jnp.int32),
            pltpu.VMEM((nb, BATCH), jnp.int32),
            pltpu.VMEM((BATCH, d), jnp.float32),
            pltpu.VMEM_SHARED((nrows, d), jnp.float32),
            pltpu.SemaphoreType.DMA,
        ],
    )
    def spmv_kernel(q_hbm, src_hbm, dst_hbm, out_hbm, src_v, dst_v, rows_v,
                    acc_sh, sem):
        c = lax.axis_index("c")
        s = lax.axis_index("s")
        wid = s * NC + c

        pltpu.sync_copy(src_hbm.at[wid], src_v)
        pltpu.sync_copy(dst_hbm.at[wid], dst_v)

        nvec = d // LANES

        def zero_body(k, carry):
            rows_v[k // nvec, pl.ds((k % nvec) * LANES, LANES)] = (
                jnp.zeros((LANES,), jnp.float32))
            return carry

        lax.fori_loop(0, BATCH * nvec, zero_body, 0)
        row0 = s * rows_per_sub
        for off, sz in chunks:
            pltpu.sync_copy(rows_v.at[pl.ds(0, sz)],
                            acc_sh.at[pl.ds(row0 + off, sz)])
        plsc.subcore_barrier()

        def body(j, carry):
            pltpu.async_copy(q_hbm.at[src_v.at[j]], rows_v, sem).wait()
            pltpu.sync_copy(rows_v, acc_sh.at[dst_v.at[j]], add=True)
            return carry

        lax.fori_loop(0, nb, body, 0)
        plsc.subcore_barrier()

        pltpu.sync_copy(acc_sh.at[pl.ds(row0, rows_per_sub)],
                        out_hbm.at[c, pl.ds(row0, rows_per_sub)])

    return spmv_kernel


def _dinv_from_partials(degp, bn2):
    nw, nrows = degp.shape
    grid = nrows // bn2

    def body(degp_ref, dinv_ref):
        deg = jnp.sum(degp_ref[...], axis=0)
        safe = jnp.where(deg > 0.0, deg, 1.0)
        dinv_ref[...] = jnp.where(deg > 0.0, lax.rsqrt(safe), 0.0)[:, None]

    return pl.pallas_call(
        body,
        grid=(grid,),
        in_specs=[pl.BlockSpec((nw, bn2), lambda i: (0, i))],
        out_specs=pl.BlockSpec((bn2, 1), lambda i: (i, 0)),
        out_shape=jax.ShapeDtypeStruct((nrows, 1), jnp.float32),
    )(degp)


def _combine0(dinv, x, c0, bn):
    n, d = x.shape
    grid = n // bn

    def body(dinv_ref, x_ref, c_ref, q_ref, out_ref):
        xv = x_ref[...]
        q_ref[...] = dinv_ref[...] * xv
        out_ref[...] = c_ref[0, 0] * xv

    return pl.pallas_call(
        body,
        grid=(grid,),
        in_specs=[
            pl.BlockSpec((bn, 1), lambda i: (i, 0)),
            pl.BlockSpec((bn, d), lambda i: (i, 0)),
            pl.BlockSpec((1, 1), lambda i: (0, 0)),
        ],
        out_specs=[
            pl.BlockSpec((bn, d), lambda i: (i, 0)),
            pl.BlockSpec((bn, d), lambda i: (i, 0)),
        ],
        out_shape=[
            jax.ShapeDtypeStruct((n, d), jnp.float32),
            jax.ShapeDtypeStruct((n, d), jnp.float32),
        ],
    )(dinv, x, c0)


def _combine_step(acc, dinv, prev, pprev, outp, cn, kp0, kp1, kp2, bn):
    n, d = prev.shape
    grid = n // bn

    def body(acc_ref, dinv_ref, prev_ref, pprev_ref, outp_ref, c_ref,
             pn_ref, qn_ref, outn_ref):
        dv = dinv_ref[...]
        ap = dv * (acc_ref[0] + acc_ref[1])
        pn = kp0 * prev_ref[...] - kp1 * ap - kp2 * pprev_ref[...]
        pn_ref[...] = pn
        qn_ref[...] = dv * pn
        outn_ref[...] = outp_ref[...] + c_ref[0, 0] * pn

    nrows = acc.shape[1]
    del nrows
    return pl.pallas_call(
        body,
        grid=(grid,),
        in_specs=[
            pl.BlockSpec((NC, bn, d), lambda i: (0, i, 0)),
            pl.BlockSpec((bn, 1), lambda i: (i, 0)),
            pl.BlockSpec((bn, d), lambda i: (i, 0)),
            pl.BlockSpec((bn, d), lambda i: (i, 0)),
            pl.BlockSpec((bn, d), lambda i: (i, 0)),
            pl.BlockSpec((1, 1), lambda i: (0, 0)),
        ],
        out_specs=[
            pl.BlockSpec((bn, d), lambda i: (i, 0)),
            pl.BlockSpec((bn, d), lambda i: (i, 0)),
            pl.BlockSpec((bn, d), lambda i: (i, 0)),
        ],
        out_shape=[
            jax.ShapeDtypeStruct((n, d), jnp.float32),
            jax.ShapeDtypeStruct((n, d), jnp.float32),
            jax.ShapeDtypeStruct((n, d), jnp.float32),
        ],
    )(acc, dinv, prev, pprev, outp, cn)


def kernel(x, edge_index, coeffs):
    n, d = x.shape
    e = edge_index.shape[1]
    order = coeffs.shape[0] - 1
    a, b = ALPHA, BETA

    assert e % NW == 0
    epw = e // NW
    nb = -(-epw // BATCH)
    epw_pad = nb * BATCH
    nrows = -(-(n + 1) // NS // LANES) * NS * LANES

    src = edge_index[0].reshape(NW, epw)
    dst = edge_index[1].reshape(NW, epw)
    pad = epw_pad - epw
    srcp = jnp.pad(src, ((0, 0), (0, pad)), constant_values=0)
    dstp = jnp.pad(dst, ((0, 0), (0, pad)), constant_values=n)
    src3 = srcp.reshape(NW, nb, BATCH)
    dst3 = dstp.reshape(NW, nb, BATCH)

    deg_kernel = _make_deg_kernel(nrows, epw_pad)
    spmv_kernel = _make_spmv_kernel(n, nrows, d, nb)

    bn = 1000 if n % 1000 == 0 else n
    degp = deg_kernel(dstp)
    bn2 = 1280 if nrows % 1280 == 0 else nrows
    dinv_full = _dinv_from_partials(degp, bn2)
    dinv = dinv_full[:n]
    q, out = _combine0(dinv, x, coeffs[0].reshape(1, 1), bn)

    prev, pprev = x, x
    for step in range(1, order + 1):
        nn = float(step)
        if step == 1:
            kp0 = 0.5 * (a - b)
            kp1 = 0.5 * (a + b + 2.0)
            kp2 = 0.0
        else:
            denom = 2.0 * nn * (nn + a + b) * (2.0 * nn + a + b - 2.0)
            kp0 = (2.0 * nn + a + b - 1.0) * (a * a - b * b) / denom
            kp1 = ((2.0 * nn + a + b - 1.0) * (2.0 * nn + a + b)
                   * (2.0 * nn + a + b - 2.0)) / denom
            kp2 = 2.0 * (nn + a - 1.0) * (nn + b - 1.0) * (2.0 * nn + a + b) / denom
        acc = spmv_kernel(q, src3, dst3)
        pn, qn, out = _combine_step(acc, dinv, prev, pprev, out,
                                    coeffs[step].reshape(1, 1),
                                    kp0, kp1, kp2, bn)
        prev, pprev, q = pn, prev, qn
    return out
